# baseline (speedup 1.0000x reference)
import functools

import jax
import jax.numpy as jnp
from jax import lax
from jax.experimental import pallas as pl
from jax.experimental.pallas import tpu as pltpu
from jax.experimental.pallas import tpu_sc as plsc

N = 10000
E = 320000
D = 128
R = 8
NPAD = 10240
NC = 2
NS = 16
NW = NC * NS
CH = 128
NCHUNK = E // CH
CNT_W = 16
CNT_ROWS = 82048
CNT_TILE = CNT_ROWS // NS
DUMP_KEY = R * NPAD
EC = 327680
CROWS = EC // CH

_mesh = plsc.VectorSubcoreMesh(core_axis_name="c", subcore_axis_name="s")
_sc_params = pltpu.CompilerParams(use_tc_tiling_on_sc=False)


@functools.partial(
    pl.kernel,
    out_type=jax.ShapeDtypeStruct((NC, CNT_ROWS, CNT_W), jnp.float32),
    mesh=_mesh,
    compiler_params=_sc_params,
    scratch_types=[
        pltpu.VMEM((8, CH), jnp.int32),
        pltpu.VMEM((CH, CNT_W), jnp.float32),
        pltpu.VMEM_SHARED((CNT_ROWS, CNT_W), jnp.float32),
    ],
)
def _counts_kernel(key2p_h, ones_h, zeros_h, cnt_h, kidx, onesb, cntsp):
    cid = lax.axis_index("c")
    sid = lax.axis_index("s")
    pltpu.sync_copy(ones_h, onesb)
    pltpu.sync_copy(zeros_h, cntsp.at[pl.ds(sid * CNT_TILE, CNT_TILE)])
    plsc.subcore_barrier()
    base = cid * (CROWS // NC) + sid * (CROWS // NC // NS)

    def grp(gi, carry):
        row = base + gi * 8
        pltpu.sync_copy(key2p_h.at[pl.ds(row, 8)], kidx)
        for j in range(8):
            pltpu.sync_copy(onesb, cntsp.at[kidx.at[j]], add=True)
        return carry

    lax.fori_loop(0, CROWS // NC // NS // 8, grp, 0)
    plsc.subcore_barrier()
    pltpu.sync_copy(cntsp.at[pl.ds(sid * CNT_TILE, CNT_TILE)],
                    cnt_h.at[cid, pl.ds(sid * CNT_TILE, CNT_TILE)])


@functools.partial(
    pl.kernel,
    out_type=(jax.ShapeDtypeStruct((EC, CNT_W), jnp.float32),
              jax.ShapeDtypeStruct((NPAD, D), jnp.float32)),
    mesh=_mesh,
    compiler_params=_sc_params,
    scratch_types=[
        pltpu.VMEM((8, CH), jnp.int32),
        pltpu.VMEM((CH, CNT_W), jnp.float32),
        pltpu.VMEM((CH, CNT_W), jnp.float32),
        pltpu.VMEM((CH, CNT_W), jnp.float32),
        pltpu.VMEM((CH, CNT_W), jnp.float32),
        pltpu.VMEM((8 * CH, CNT_W), jnp.float32),
        pltpu.VMEM((64,), jnp.int32),
        pltpu.VMEM((64, D), jnp.float32),
        pltpu.SemaphoreType.DMA,
        pltpu.SemaphoreType.DMA,
        pltpu.SemaphoreType.DMA,
        pltpu.SemaphoreType.DMA,
        pltpu.SemaphoreType.DMA,
    ],
)
def _prep_kernel(cnt0_h, cnt1_h, key2p_h, xp_h, embed_h, w_h, h0_h,
                 kb, cr0a, cr0b, cr1a, cr1b, wb8, xib, hrows,
                 s0a, s0b, s1a, s1b, sem):
    cid = lax.axis_index("c")
    sid = lax.axis_index("s")
    gw = sid * NC + cid
    cr0 = (cr0a, cr0b)
    cr1 = (cr1a, cr1b)
    sg0 = (s0a, s0b)
    sg1 = (s1a, s1b)

    def group(gi, carry):
        chg = gw * 80 + gi * 8
        pltpu.sync_copy(key2p_h.at[pl.ds(chg, 8)], kb)
        gat = [None, None]
        gat[0] = (pltpu.async_copy(cnt0_h.at[kb.at[0]], cr0[0], sg0[0]),
                  pltpu.async_copy(cnt1_h.at[kb.at[0]], cr1[0], sg1[0]))
        for b in range(8):
            p = b & 1
            gat[p][0].wait()
            gat[p][1].wait()
            if b < 7:
                q = 1 - p
                gat[q] = (
                    pltpu.async_copy(cnt0_h.at[kb.at[b + 1]], cr0[q], sg0[q]),
                    pltpu.async_copy(cnt1_h.at[kb.at[b + 1]], cr1[q], sg1[q]))

            def wrow(i, c3):
                wb8[b * CH + i, pl.ds(0, CNT_W)] = 1.0 / (
                    cr0[p][i, pl.ds(0, CNT_W)] + cr1[p][i, pl.ds(0, CNT_W)])
                return c3

            lax.fori_loop(0, CH, wrow, 0, unroll=4)

        pltpu.sync_copy(wb8, w_h.at[pl.ds(pl.multiple_of(chg * CH, 1024),
                                          8 * CH)])
        return carry

    lax.fori_loop(0, 10, group, 0)
    for c in range(NPAD // NW // 64):
        o = gw * (NPAD // NW) + c * 64
        pltpu.sync_copy(xp_h.at[pl.ds(o, 64)], xib)
        pltpu.async_copy(embed_h.at[xib], hrows, sem).wait()
        pltpu.sync_copy(hrows, h0_h.at[pl.ds(o, 64)])


@functools.partial(
    pl.kernel,
    out_type=jax.ShapeDtypeStruct((NC, NPAD, D), jnp.float32),
    mesh=_mesh,
    compiler_params=_sc_params,
    scratch_types=[
        pltpu.VMEM((8 * CH,), jnp.int32),
        pltpu.VMEM((8, CH), jnp.int32),
        pltpu.VMEM((CH, CNT_W), jnp.float32),
        pltpu.VMEM((CH, CNT_W), jnp.float32),
        pltpu.VMEM((CH, D), jnp.float32),
        pltpu.VMEM((CH, D), jnp.float32),
        pltpu.VMEM_SHARED((NPAD, D), jnp.float32),
        pltpu.SemaphoreType.DMA,
        pltpu.SemaphoreType.DMA,
        pltpu.SemaphoreType.DMA,
        pltpu.SemaphoreType.DMA,
        pltpu.SemaphoreType.DMA,
        pltpu.SemaphoreType.DMA,
    ],
)
def _edge_kernel(zz_h, g_h, dst2d_h, w_h, zeros_h, msg_h,
                 gb, db8, wb0, wb1, rows0, rows1, acc,
                 gs0, gs1, ws0, ws1, ss0, ss1):
    cid = lax.axis_index("c")
    sid = lax.axis_index("s")
    gw = sid * NC + cid
    rows = (rows0, rows1)
    wb = (wb0, wb1)
    gsem = (gs0, gs1)
    wsem = (ws0, ws1)
    ssem = (ss0, ss1)
    for c in range(NPAD // NS // CH):
        pltpu.sync_copy(zeros_h, acc.at[pl.ds(sid * (NPAD // NS) + c * CH, CH)])
    plsc.subcore_barrier()

    def group(gi, carry):
        chg = gw * 80 + gi * 8
        off = pl.multiple_of(chg * CH, 1024)
        pltpu.sync_copy(g_h.at[pl.ds(off, 8 * CH)], gb)
        pltpu.sync_copy(dst2d_h.at[pl.ds(chg, 8)], db8)
        gat = [None, None]
        wat = [None, None]
        scat = [None, None]
        gat[0] = pltpu.async_copy(zz_h.at[gb.at[pl.ds(0, CH)]], rows[0],
                                  gsem[0])
        wat[0] = pltpu.async_copy(w_h.at[pl.ds(off, CH)], wb[0], wsem[0])
        for b in range(8):
            p = b & 1
            gat[p].wait()
            wat[p].wait()
            if b < 7:
                q = 1 - p
                if scat[q] is not None:
                    scat[q].wait()
                gat[q] = pltpu.async_copy(
                    zz_h.at[gb.at[pl.ds((b + 1) * CH, CH)]], rows[q], gsem[q])
                wat[q] = pltpu.async_copy(
                    w_h.at[pl.ds(off + (b + 1) * CH, CH)], wb[q], wsem[q])

            def scale(k, c2, _p=p):
                wv = wb[_p][k, pl.ds(0, CNT_W)]
                for j in range(D // 16):
                    rows[_p][k, pl.ds(j * 16, 16)] = (
                        rows[_p][k, pl.ds(j * 16, 16)] * wv)
                return c2

            lax.fori_loop(0, CH, scale, 0, unroll=4)

            scat[p] = pltpu.async_copy(rows[p], acc.at[db8.at[b]], ssem[p],
                                       add=True)
        scat[0].wait()
        scat[1].wait()
        return carry

    lax.fori_loop(0, 10, group, 0)
    plsc.subcore_barrier()
    for c in range(NPAD // NS // CH):
        sl = sid * (NPAD // NS) + c * CH
        pltpu.sync_copy(acc.at[pl.ds(sl, CH)], msg_h.at[cid, pl.ds(sl, CH)])


def _mm_body(h_ref, w_ref, o_ref):
    o_ref[0] = jnp.dot(h_ref[...], w_ref[0],
                       preferred_element_type=jnp.float32)


def _matmul(h, wall):
    BM = 512
    return pl.pallas_call(
        _mm_body,
        grid=(R + 1, NPAD // BM),
        in_specs=[pl.BlockSpec((BM, D), lambda r, i: (i, 0)),
                  pl.BlockSpec((1, D, D), lambda r, i: (r, 0, 0))],
        out_specs=pl.BlockSpec((1, BM, D), lambda r, i: (r, i, 0)),
        out_shape=jax.ShapeDtypeStruct((R + 1, NPAD, D), jnp.float32),
    )(h, wall)


def _combine(zz, msg, bias, do_relu):
    BM = 512

    def body(z_ref, m_ref, b_ref, o_ref):
        s = z_ref[0] + m_ref[0] + m_ref[1] + b_ref[...]
        if do_relu:
            s = jnp.maximum(s, 0.0)
        o_ref[...] = s

    return pl.pallas_call(
        body,
        grid=(NPAD // BM,),
        in_specs=[pl.BlockSpec((1, BM, D), lambda i: (0, i, 0)),
                  pl.BlockSpec((NC, BM, D), lambda i: (0, i, 0)),
                  pl.BlockSpec((1, D), lambda i: (0, 0))],
        out_specs=pl.BlockSpec((BM, D), lambda i: (i, 0)),
        out_shape=jax.ShapeDtypeStruct((NPAD, D), jnp.float32),
    )(zz, msg, bias.reshape(1, D))


def kernel(x, edge_index, edge_type, embed_weight, W1, root1, b1,
           W2, root2, b2):
    src = edge_index[0].astype(jnp.int32)
    dst = edge_index[1].astype(jnp.int32)
    et = edge_type.astype(jnp.int32)
    key2 = et * NPAD + dst
    key2p = jnp.pad(key2, (0, EC - E),
                    constant_values=DUMP_KEY).reshape(CROWS, CH)
    g = jnp.pad((et + 1) * NPAD + src, (0, EC - E))
    dst2d = jnp.pad(dst, (0, EC - E),
                    constant_values=NPAD - 1).reshape(CROWS, CH)
    xp = jnp.pad(x.astype(jnp.int32), (0, NPAD - N))
    ones_in = jnp.ones((CH, CNT_W), jnp.float32)
    zeros_cnt = jnp.zeros((CNT_TILE, CNT_W), jnp.float32)
    zeros_acc = jnp.zeros((CH, D), jnp.float32)

    cnt = _counts_kernel(key2p, ones_in, zeros_cnt)
    w, h0 = _prep_kernel(cnt[0], cnt[1], key2p, xp, embed_weight)

    wall1 = jnp.concatenate([root1[None], W1], axis=0)
    wall2 = jnp.concatenate([root2[None], W2], axis=0)

    zz1 = _matmul(h0, wall1)
    msg1 = _edge_kernel(zz1.reshape((R + 1) * NPAD, D), g, dst2d, w, zeros_acc)
    h1 = _combine(zz1, msg1, b1, True)

    zz2 = _matmul(h1, wall2)
    msg2 = _edge_kernel(zz2.reshape((R + 1) * NPAD, D), g, dst2d, w, zeros_acc)
    out = _combine(zz2, msg2, b2, False)
    return out[:N]


# speedup vs baseline: 1.6194x; 1.6194x over previous
import functools

import jax
import jax.numpy as jnp
from jax import lax
from jax.experimental import pallas as pl
from jax.experimental.pallas import tpu as pltpu
from jax.experimental.pallas import tpu_sc as plsc

N = 10000
E = 320000
D = 128
R = 8
NPAD = 10240
NC = 2
NS = 16
NW = NC * NS
CH = 128
NCHUNK = E // CH
CNT_W = 16
CNT_ROWS = 82048
CNT_TILE = CNT_ROWS // NS
DUMP_KEY = R * NPAD
EC = 327680
CROWS = EC // CH

_mesh = plsc.VectorSubcoreMesh(core_axis_name="c", subcore_axis_name="s")
_sc_params = pltpu.CompilerParams(use_tc_tiling_on_sc=False)


@functools.partial(
    pl.kernel,
    out_type=jax.ShapeDtypeStruct((NC, CNT_ROWS, CNT_W), jnp.float32),
    mesh=_mesh,
    compiler_params=_sc_params,
    scratch_types=[
        pltpu.VMEM((8, CH), jnp.int32),
        pltpu.VMEM((CH, CNT_W), jnp.float32),
        pltpu.VMEM_SHARED((CNT_ROWS, CNT_W), jnp.float32),
    ],
)
def _counts_kernel(key2p_h, ones_h, zeros_h, cnt_h, kidx, onesb, cntsp):
    cid = lax.axis_index("c")
    sid = lax.axis_index("s")
    pltpu.sync_copy(ones_h, onesb)
    pltpu.sync_copy(zeros_h, cntsp.at[pl.ds(sid * CNT_TILE, CNT_TILE)])
    plsc.subcore_barrier()
    base = cid * (CROWS // NC) + sid * (CROWS // NC // NS)

    def grp(gi, carry):
        row = base + gi * 8
        pltpu.sync_copy(key2p_h.at[pl.ds(row, 8)], kidx)
        for j in range(8):
            pltpu.sync_copy(onesb, cntsp.at[kidx.at[j]], add=True)
        return carry

    lax.fori_loop(0, CROWS // NC // NS // 8, grp, 0)
    plsc.subcore_barrier()
    pltpu.sync_copy(cntsp.at[pl.ds(sid * CNT_TILE, CNT_TILE)],
                    cnt_h.at[cid, pl.ds(sid * CNT_TILE, CNT_TILE)])


@functools.partial(
    pl.kernel,
    out_type=(jax.ShapeDtypeStruct((EC, CNT_W), jnp.float32),
              jax.ShapeDtypeStruct((NPAD, D), jnp.float32)),
    mesh=_mesh,
    compiler_params=_sc_params,
    scratch_types=[
        pltpu.VMEM((8, CH), jnp.int32),
        pltpu.VMEM((CH, CNT_W), jnp.float32),
        pltpu.VMEM((CH, CNT_W), jnp.float32),
        pltpu.VMEM((CH, CNT_W), jnp.float32),
        pltpu.VMEM((CH, CNT_W), jnp.float32),
        pltpu.VMEM((8 * CH, CNT_W), jnp.float32),
        pltpu.VMEM((64,), jnp.int32),
        pltpu.VMEM((64, D), jnp.float32),
        pltpu.SemaphoreType.DMA,
        pltpu.SemaphoreType.DMA,
        pltpu.SemaphoreType.DMA,
        pltpu.SemaphoreType.DMA,
        pltpu.SemaphoreType.DMA,
    ],
)
def _prep_kernel(cnt0_h, cnt1_h, key2p_h, xp_h, embed_h, w_h, h0_h,
                 kb, cr0a, cr0b, cr1a, cr1b, wb8, xib, hrows,
                 s0a, s0b, s1a, s1b, sem):
    cid = lax.axis_index("c")
    sid = lax.axis_index("s")
    gw = sid * NC + cid
    cr0 = (cr0a, cr0b)
    cr1 = (cr1a, cr1b)
    sg0 = (s0a, s0b)
    sg1 = (s1a, s1b)

    def group(gi, carry):
        chg = gw * 80 + gi * 8
        pltpu.sync_copy(key2p_h.at[pl.ds(chg, 8)], kb)
        gat = [None, None]
        gat[0] = (pltpu.async_copy(cnt0_h.at[kb.at[0]], cr0[0], sg0[0]),
                  pltpu.async_copy(cnt1_h.at[kb.at[0]], cr1[0], sg1[0]))
        for b in range(8):
            p = b & 1
            gat[p][0].wait()
            gat[p][1].wait()
            if b < 7:
                q = 1 - p
                gat[q] = (
                    pltpu.async_copy(cnt0_h.at[kb.at[b + 1]], cr0[q], sg0[q]),
                    pltpu.async_copy(cnt1_h.at[kb.at[b + 1]], cr1[q], sg1[q]))

            def wrow(i, c3):
                wb8[b * CH + i, pl.ds(0, CNT_W)] = 1.0 / (
                    cr0[p][i, pl.ds(0, CNT_W)] + cr1[p][i, pl.ds(0, CNT_W)])
                return c3

            lax.fori_loop(0, CH, wrow, 0, unroll=4)

        pltpu.sync_copy(wb8, w_h.at[pl.ds(pl.multiple_of(chg * CH, 1024),
                                          8 * CH)])
        return carry

    lax.fori_loop(0, 10, group, 0)
    for c in range(NPAD // NW // 64):
        o = gw * (NPAD // NW) + c * 64
        pltpu.sync_copy(xp_h.at[pl.ds(o, 64)], xib)
        pltpu.async_copy(embed_h.at[xib], hrows, sem).wait()
        pltpu.sync_copy(hrows, h0_h.at[pl.ds(o, 64)])


@functools.partial(
    pl.kernel,
    out_type=jax.ShapeDtypeStruct((NC, NPAD, D), jnp.float32),
    mesh=_mesh,
    compiler_params=_sc_params,
    scratch_types=[
        pltpu.VMEM((8 * CH,), jnp.int32),
        pltpu.VMEM((8, CH), jnp.int32),
        pltpu.VMEM((CH, CNT_W), jnp.float32),
        pltpu.VMEM((CH, CNT_W), jnp.float32),
        pltpu.VMEM((CH, D), jnp.float32),
        pltpu.VMEM((CH, D), jnp.float32),
        pltpu.VMEM_SHARED((NPAD, D), jnp.float32),
        pltpu.SemaphoreType.DMA,
        pltpu.SemaphoreType.DMA,
        pltpu.SemaphoreType.DMA,
        pltpu.SemaphoreType.DMA,
        pltpu.SemaphoreType.DMA,
        pltpu.SemaphoreType.DMA,
    ],
)
def _edge_kernel(zz_h, g_h, dst2d_h, w_h, zeros_h, msg_h,
                 gb, db8, wb0, wb1, rows0, rows1, acc,
                 gs0, gs1, ws0, ws1, ss0, ss1):
    cid = lax.axis_index("c")
    sid = lax.axis_index("s")
    gw = sid * NC + cid
    rows = (rows0, rows1)
    wb = (wb0, wb1)
    gsem = (gs0, gs1)
    wsem = (ws0, ws1)
    ssem = (ss0, ss1)
    for c in range(NPAD // NS // CH):
        pltpu.sync_copy(zeros_h, acc.at[pl.ds(sid * (NPAD // NS) + c * CH, CH)])
    plsc.subcore_barrier()

    def group(gi, carry):
        chg = gw * 80 + gi * 8
        off = pl.multiple_of(chg * CH, 1024)
        pltpu.sync_copy(g_h.at[pl.ds(off, 8 * CH)], gb)
        pltpu.sync_copy(dst2d_h.at[pl.ds(chg, 8)], db8)
        gat = [None, None]
        wat = [None, None]
        scat = [None, None]
        gat[0] = pltpu.async_copy(zz_h.at[gb.at[pl.ds(0, CH)]], rows[0],
                                  gsem[0])
        wat[0] = pltpu.async_copy(w_h.at[pl.ds(off, CH)], wb[0], wsem[0])
        for b in range(8):
            p = b & 1
            gat[p].wait()
            wat[p].wait()
            if b < 7:
                q = 1 - p
                if scat[q] is not None:
                    scat[q].wait()
                gat[q] = pltpu.async_copy(
                    zz_h.at[gb.at[pl.ds((b + 1) * CH, CH)]], rows[q], gsem[q])
                wat[q] = pltpu.async_copy(
                    w_h.at[pl.ds(off + (b + 1) * CH, CH)], wb[q], wsem[q])

            def scale(k, c2, _p=p):
                wv = wb[_p][k, pl.ds(0, CNT_W)]
                for j in range(D // 16):
                    rows[_p][k, pl.ds(j * 16, 16)] = (
                        rows[_p][k, pl.ds(j * 16, 16)] * wv)
                return c2

            lax.fori_loop(0, CH, scale, 0, unroll=4)

            scat[p] = pltpu.async_copy(rows[p], acc.at[db8.at[b]], ssem[p],
                                       add=True)
        scat[0].wait()
        scat[1].wait()
        return carry

    lax.fori_loop(0, 10, group, 0)
    plsc.subcore_barrier()
    for c in range(NPAD // NS // CH):
        sl = sid * (NPAD // NS) + c * CH
        pltpu.sync_copy(acc.at[pl.ds(sl, CH)], msg_h.at[cid, pl.ds(sl, CH)])


def _mm_body(h_ref, w_ref, o_ref):
    o_ref[0] = jnp.dot(h_ref[...], w_ref[0],
                       preferred_element_type=jnp.float32)


def _matmul(h, wall):
    BM = 512
    return pl.pallas_call(
        _mm_body,
        grid=(R + 1, NPAD // BM),
        in_specs=[pl.BlockSpec((BM, D), lambda r, i: (i, 0)),
                  pl.BlockSpec((1, D, D), lambda r, i: (r, 0, 0))],
        out_specs=pl.BlockSpec((1, BM, D), lambda r, i: (r, i, 0)),
        out_shape=jax.ShapeDtypeStruct((R + 1, NPAD, D), jnp.float32),
    )(h, wall)


def _combine(zz, msg, bias, do_relu):
    BM = 512

    def body(z_ref, m_ref, b_ref, o_ref):
        s = z_ref[0] + m_ref[0] + m_ref[1] + b_ref[...]
        if do_relu:
            s = jnp.maximum(s, 0.0)
        o_ref[...] = s

    return pl.pallas_call(
        body,
        grid=(NPAD // BM,),
        in_specs=[pl.BlockSpec((1, BM, D), lambda i: (0, i, 0)),
                  pl.BlockSpec((NC, BM, D), lambda i: (0, i, 0)),
                  pl.BlockSpec((1, D), lambda i: (0, 0))],
        out_specs=pl.BlockSpec((BM, D), lambda i: (i, 0)),
        out_shape=jax.ShapeDtypeStruct((NPAD, D), jnp.float32),
    )(zz, msg, bias.reshape(1, D))


def kernel(x, edge_index, edge_type, embed_weight, W1, root1, b1,
           W2, root2, b2):
    src = edge_index[0].astype(jnp.int32)
    dst = edge_index[1].astype(jnp.int32)
    et = edge_type.astype(jnp.int32)
    key2 = et * NPAD + dst
    key2p = jnp.pad(key2, (0, EC - E),
                    constant_values=DUMP_KEY).reshape(CROWS, CH)
    pad_ar = jnp.arange(EC - E, dtype=jnp.int32)
    g = jnp.concatenate([(et + 1) * NPAD + src, pad_ar % NPAD])
    dst2d = jnp.concatenate([dst, N + pad_ar % (NPAD - N)]).reshape(CROWS, CH)
    xp = jnp.pad(x.astype(jnp.int32), (0, NPAD - N))
    ones_in = jnp.ones((CH, CNT_W), jnp.float32)
    zeros_cnt = jnp.zeros((CNT_TILE, CNT_W), jnp.float32)
    zeros_acc = jnp.zeros((CH, D), jnp.float32)

    cnt = _counts_kernel(key2p, ones_in, zeros_cnt)
    w, h0 = _prep_kernel(cnt[0], cnt[1], key2p, xp, embed_weight)

    wall1 = jnp.concatenate([root1[None], W1], axis=0)
    wall2 = jnp.concatenate([root2[None], W2], axis=0)

    zz1 = _matmul(h0, wall1)
    msg1 = _edge_kernel(zz1.reshape((R + 1) * NPAD, D), g, dst2d, w, zeros_acc)
    h1 = _combine(zz1, msg1, b1, True)

    zz2 = _matmul(h1, wall2)
    msg2 = _edge_kernel(zz2.reshape((R + 1) * NPAD, D), g, dst2d, w, zeros_acc)
    out = _combine(zz2, msg2, b2, False)
    return out[:N]


# speedup vs baseline: 2.1218x; 1.3102x over previous
import functools

import jax
import jax.numpy as jnp
from jax import lax
from jax.experimental import pallas as pl
from jax.experimental.pallas import tpu as pltpu
from jax.experimental.pallas import tpu_sc as plsc

N = 10000
E = 320000
D = 128
R = 8
NPAD = 10240
NC = 2
NS = 16
NW = NC * NS
CH = 128
CNT_W = 16
CNT_ROWS = R * NPAD
CNT_TILE = CNT_ROWS // NS
DUMP_KEY = N
EC = 327680
CROWS = EC // CH

_mesh = plsc.VectorSubcoreMesh(core_axis_name="c", subcore_axis_name="s")
_sc_params = pltpu.CompilerParams(use_tc_tiling_on_sc=False)


@functools.partial(
    pl.kernel,
    out_type=(jax.ShapeDtypeStruct((CNT_ROWS, CNT_W), jnp.float32),
              jax.ShapeDtypeStruct((NPAD, D), jnp.float32)),
    mesh=_mesh,
    compiler_params=_sc_params,
    scratch_types=[
        pltpu.VMEM((8, CH), jnp.int32),
        pltpu.VMEM((CH, CNT_W), jnp.float32),
        pltpu.VMEM((CH, CNT_W), jnp.float32),
        pltpu.VMEM((64,), jnp.int32),
        pltpu.VMEM((64, D), jnp.float32),
        pltpu.VMEM_SHARED((CNT_ROWS, CNT_W), jnp.float32),
        pltpu.SemaphoreType.DMA,
    ],
)
def _counts_kernel(key2p_h, ones_h, zeros_h, xp_h, embed_h, inv_h, h0_h,
                   kidx, onesb, crb, xib, hrows, cntsp, sem):
    cid = lax.axis_index("c")
    sid = lax.axis_index("s")
    gw = sid * NC + cid
    pltpu.sync_copy(ones_h, onesb)
    pltpu.sync_copy(zeros_h, cntsp.at[pl.ds(sid * CNT_TILE, CNT_TILE)])
    plsc.subcore_barrier()
    base = sid * (CROWS // NS)

    def grp(gi, carry):
        row = base + gi * 8
        pltpu.sync_copy(key2p_h.at[pl.ds(row, 8)], kidx)
        for j in range(8):
            pltpu.sync_copy(onesb, cntsp.at[kidx.at[j]], add=True)
        return carry

    lax.fori_loop(0, CROWS // NS // 8, grp, 0)
    plsc.subcore_barrier()
    rb = gw * (CNT_ROWS // NW)

    def recip(ci, carry):
        row = rb + ci * CH
        pltpu.sync_copy(cntsp.at[pl.ds(row, CH)], crb)

        def inv1(i, c2):
            crb[i, pl.ds(0, CNT_W)] = 1.0 / crb[i, pl.ds(0, CNT_W)]
            return c2

        lax.fori_loop(0, CH, inv1, 0, unroll=4)
        pltpu.sync_copy(crb, inv_h.at[pl.ds(row, CH)])
        return carry

    lax.fori_loop(0, CNT_ROWS // NW // CH, recip, 0)
    for c in range(NPAD // NW // 64):
        o = gw * (NPAD // NW) + c * 64
        pltpu.sync_copy(xp_h.at[pl.ds(o, 64)], xib)
        pltpu.async_copy(embed_h.at[xib], hrows, sem).wait()
        pltpu.sync_copy(hrows, h0_h.at[pl.ds(o, 64)])


@functools.partial(
    pl.kernel,
    out_type=jax.ShapeDtypeStruct((NC, NPAD, D), jnp.float32),
    mesh=_mesh,
    compiler_params=_sc_params,
    scratch_types=[
        pltpu.VMEM((8 * CH,), jnp.int32),
        pltpu.VMEM((8, CH), jnp.int32),
        pltpu.VMEM((8, CH), jnp.int32),
        pltpu.VMEM((CH, CNT_W), jnp.float32),
        pltpu.VMEM((CH, CNT_W), jnp.float32),
        pltpu.VMEM((CH, D), jnp.float32),
        pltpu.VMEM((CH, D), jnp.float32),
        pltpu.VMEM_SHARED((NPAD, D), jnp.float32),
        pltpu.SemaphoreType.DMA,
        pltpu.SemaphoreType.DMA,
        pltpu.SemaphoreType.DMA,
        pltpu.SemaphoreType.DMA,
        pltpu.SemaphoreType.DMA,
        pltpu.SemaphoreType.DMA,
    ],
)
def _edge_kernel(zz_h, g_h, dst2d_h, key2p_h, inv_h, zeros_h, msg_h,
                 gb, kb8, db8, wb0, wb1, rows0, rows1, acc,
                 gs0, gs1, ws0, ws1, ss0, ss1):
    cid = lax.axis_index("c")
    sid = lax.axis_index("s")
    gw = sid * NC + cid
    rows = (rows0, rows1)
    wb = (wb0, wb1)
    gsem = (gs0, gs1)
    wsem = (ws0, ws1)
    ssem = (ss0, ss1)
    for c in range(NPAD // NS // CH):
        pltpu.sync_copy(zeros_h, acc.at[pl.ds(sid * (NPAD // NS) + c * CH, CH)])
    plsc.subcore_barrier()

    def group(gi, carry):
        chg = gw * 80 + gi * 8
        off = pl.multiple_of(chg * CH, 1024)
        pltpu.sync_copy(g_h.at[pl.ds(off, 8 * CH)], gb)
        pltpu.sync_copy(key2p_h.at[pl.ds(chg, 8)], kb8)
        pltpu.sync_copy(dst2d_h.at[pl.ds(chg, 8)], db8)
        gat = [None, None]
        wat = [None, None]
        scat = [None, None]
        gat[0] = pltpu.async_copy(zz_h.at[gb.at[pl.ds(0, CH)]], rows[0],
                                  gsem[0])
        wat[0] = pltpu.async_copy(inv_h.at[kb8.at[0]], wb[0], wsem[0])
        for b in range(8):
            p = b & 1
            gat[p].wait()
            wat[p].wait()
            if b < 7:
                q = 1 - p
                if scat[q] is not None:
                    scat[q].wait()
                gat[q] = pltpu.async_copy(
                    zz_h.at[gb.at[pl.ds((b + 1) * CH, CH)]], rows[q], gsem[q])
                wat[q] = pltpu.async_copy(inv_h.at[kb8.at[b + 1]], wb[q],
                                          wsem[q])

            def scale(k, c2, _p=p):
                wv = wb[_p][k, pl.ds(0, CNT_W)]
                for j in range(D // 16):
                    rows[_p][k, pl.ds(j * 16, 16)] = (
                        rows[_p][k, pl.ds(j * 16, 16)] * wv)
                return c2

            lax.fori_loop(0, CH, scale, 0, unroll=4)

            scat[p] = pltpu.async_copy(rows[p], acc.at[db8.at[b]], ssem[p],
                                       add=True)
        scat[0].wait()
        scat[1].wait()
        return carry

    lax.fori_loop(0, 10, group, 0)
    plsc.subcore_barrier()
    for c in range(NPAD // NS // CH):
        sl = sid * (NPAD // NS) + c * CH
        pltpu.sync_copy(acc.at[pl.ds(sl, CH)], msg_h.at[cid, pl.ds(sl, CH)])


def _mm_body(h_ref, w_ref, o_ref):
    o_ref[0] = jnp.dot(h_ref[...], w_ref[0],
                       preferred_element_type=jnp.float32)


def _matmul(h, wall):
    BM = 512
    return pl.pallas_call(
        _mm_body,
        grid=(R + 1, NPAD // BM),
        in_specs=[pl.BlockSpec((BM, D), lambda r, i: (i, 0)),
                  pl.BlockSpec((1, D, D), lambda r, i: (r, 0, 0))],
        out_specs=pl.BlockSpec((1, BM, D), lambda r, i: (r, i, 0)),
        out_shape=jax.ShapeDtypeStruct((R + 1, NPAD, D), jnp.float32),
    )(h, wall)


def _mmc_body(z_ref, m_ref, b_ref, w_ref, o_ref):
    h = z_ref[0] + m_ref[0] + m_ref[1] + b_ref[...]
    h = jnp.maximum(h, 0.0)
    o_ref[0] = jnp.dot(h, w_ref[0], preferred_element_type=jnp.float32)


def _matmul_combine(zz, msg, bias, wall):
    BM = 512
    return pl.pallas_call(
        _mmc_body,
        grid=(R + 1, NPAD // BM),
        in_specs=[pl.BlockSpec((1, BM, D), lambda r, i: (0, i, 0)),
                  pl.BlockSpec((NC, BM, D), lambda r, i: (0, i, 0)),
                  pl.BlockSpec((1, D), lambda r, i: (0, 0)),
                  pl.BlockSpec((1, D, D), lambda r, i: (r, 0, 0))],
        out_specs=pl.BlockSpec((1, BM, D), lambda r, i: (r, i, 0)),
        out_shape=jax.ShapeDtypeStruct((R + 1, NPAD, D), jnp.float32),
    )(zz, msg, bias.reshape(1, D), wall)


def _combine(zz, msg, bias):
    BM = 512

    def body(z_ref, m_ref, b_ref, o_ref):
        o_ref[...] = z_ref[0] + m_ref[0] + m_ref[1] + b_ref[...]

    return pl.pallas_call(
        body,
        grid=(NPAD // BM,),
        in_specs=[pl.BlockSpec((1, BM, D), lambda i: (0, i, 0)),
                  pl.BlockSpec((NC, BM, D), lambda i: (0, i, 0)),
                  pl.BlockSpec((1, D), lambda i: (0, 0))],
        out_specs=pl.BlockSpec((BM, D), lambda i: (i, 0)),
        out_shape=jax.ShapeDtypeStruct((NPAD, D), jnp.float32),
    )(zz, msg, bias.reshape(1, D))


def kernel(x, edge_index, edge_type, embed_weight, W1, root1, b1,
           W2, root2, b2):
    src = edge_index[0].astype(jnp.int32)
    dst = edge_index[1].astype(jnp.int32)
    et = edge_type.astype(jnp.int32)
    key2p = jnp.pad(et * NPAD + dst, (0, EC - E),
                    constant_values=DUMP_KEY).reshape(CROWS, CH)
    pad_ar = jnp.arange(EC - E, dtype=jnp.int32)
    g = jnp.concatenate([(et + 1) * NPAD + src, pad_ar % NPAD])
    dst2d = jnp.concatenate([dst, N + pad_ar % (NPAD - N)]).reshape(CROWS, CH)
    xp = jnp.pad(x.astype(jnp.int32), (0, NPAD - N))
    ones_in = jnp.ones((CH, CNT_W), jnp.float32)
    zeros_cnt = jnp.zeros((CNT_TILE, CNT_W), jnp.float32)
    zeros_acc = jnp.zeros((CH, D), jnp.float32)

    inv, h0 = _counts_kernel(key2p, ones_in, zeros_cnt, xp, embed_weight)

    wall1 = jnp.concatenate([root1[None], W1], axis=0)
    wall2 = jnp.concatenate([root2[None], W2], axis=0)

    zz1 = _matmul(h0, wall1)
    msg1 = _edge_kernel(zz1.reshape((R + 1) * NPAD, D), g, dst2d, key2p, inv,
                        zeros_acc)
    zz2 = _matmul_combine(zz1, msg1, b1, wall2)
    msg2 = _edge_kernel(zz2.reshape((R + 1) * NPAD, D), g, dst2d, key2p, inv,
                        zeros_acc)
    out = _combine(zz2, msg2, b2)
    return out[:N]
